# baseline (speedup 1.0000x reference)
import functools

import jax
import jax.numpy as jnp
from jax import lax
from jax.experimental import pallas as pl
from jax.experimental.pallas import tpu as pltpu
from jax.experimental.pallas import tpu_sc as plsc

K = 8
V = 2048
D = 128
B = 16
T = 2048
NW = 32
P = B * T
PPW = P // NW
C = 128
NCH = PPW // C
WPB = T // PPW
NB = 6
PF = 4


def _make_kernel():
  mesh = plsc.VectorSubcoreMesh(core_axis_name="c", subcore_axis_name="s")

  @functools.partial(
      pl.kernel,
      mesh=mesh,
      out_type=jax.ShapeDtypeStruct((P, D), jnp.float32),
      scratch_types=[pltpu.VMEM((K, PPW), jnp.int32)]
      + [pltpu.VMEM((C, D), jnp.float32)] * NB
      + [pltpu.SemaphoreType.DMA] * (3 * NCH),
  )
  def k(codes_hbm, tab_hbm, out_hbm, idx_v, *bufs):
    acc = bufs[:NB]
    sg = bufs[NB:NB + NCH]
    sa = bufs[NB + NCH:NB + 2 * NCH]
    so = bufs[NB + 2 * NCH:]
    wid = lax.axis_index("s") * 2 + lax.axis_index("c")
    b = wid // WPB
    half = wid % WPB
    pltpu.sync_copy(codes_hbm.at[b, :, pl.ds(half * PPW, PPW)], idx_v)

    def islice(kk, ci):
      return idx_v.at[kk, pl.ds(ci * C, C)]

    def plain(ci):
      pltpu.async_copy(tab_hbm.at[0].at[islice(0, ci)], acc[ci % NB], sg[ci])

    def fire_out(ci):
      pltpu.async_copy(acc[ci % NB],
                       out_hbm.at[pl.ds(wid * PPW + ci * C, C)], so[ci])

    def drain_out(ci):
      pltpu.make_async_copy(acc[ci % NB], out_hbm.at[pl.ds(0, C)],
                            so[ci]).wait()

    for ci in range(PF):
      plain(ci)
    prev_adds = None
    drained = 0
    for ci in range(NCH):
      pltpu.make_async_copy(
          tab_hbm.at[0].at[islice(0, ci)], acc[ci % NB], sg[ci]).wait()
      adds = [
          pltpu.async_copy(tab_hbm.at[kk].at[islice(kk, ci)], acc[ci % NB],
                           sa[ci], add=True)
          for kk in range(1, K)
      ]
      if prev_adds is not None:
        for cp in prev_adds:
          cp.wait()
        fire_out(ci - 1)
      prev_adds = adds
      if ci + PF < NCH:
        if ci >= 2:
          drain_out(ci - 2)
          drained = ci - 1
        plain(ci + PF)
    for cp in prev_adds:
      cp.wait()
    fire_out(NCH - 1)
    for ci in range(drained, NCH):
      drain_out(ci)

  return k


_sc_kernel = _make_kernel()


def kernel(codes, tables):
  out = _sc_kernel(codes.astype(jnp.int32), tables)
  return out.reshape(B, T, D)


# speedup vs baseline: 1.0035x; 1.0035x over previous
import functools

import jax
import jax.numpy as jnp
from jax import lax
from jax.experimental import pallas as pl
from jax.experimental.pallas import tpu as pltpu
from jax.experimental.pallas import tpu_sc as plsc

K = 8
V = 2048
D = 128
B = 16
T = 2048
NW = 32
P = B * T
PPW = P // NW
C = 128
NCH = PPW // C
WPB = T // PPW
NB = 6
PF = 4


def _make_kernel():
  mesh = plsc.VectorSubcoreMesh(core_axis_name="c", subcore_axis_name="s")

  @functools.partial(
      pl.kernel,
      mesh=mesh,
      out_type=jax.ShapeDtypeStruct((P, D), jnp.float32),
      scratch_types=[pltpu.VMEM((K, PPW), jnp.int32)]
      + [pltpu.VMEM((C, D), jnp.float32)] * NB
      + [pltpu.SemaphoreType.DMA] * (3 * NCH + 2),
  )
  def k(codes_hbm, tab_hbm, out_hbm, idx_v, *bufs):
    acc = bufs[:NB]
    sg = bufs[NB:NB + NCH]
    sa = bufs[NB + NCH:NB + 2 * NCH]
    so = bufs[NB + 2 * NCH:NB + 3 * NCH]
    si0, sir = bufs[NB + 3 * NCH:]
    wid = lax.axis_index("s") * 2 + lax.axis_index("c")
    b = wid // WPB
    half = wid % WPB
    src = codes_hbm.at[b, :, pl.ds(half * PPW, PPW)]
    c0 = pltpu.async_copy(src.at[0], idx_v.at[0], si0)
    cr = pltpu.async_copy(src.at[pl.ds(1, K - 1)],
                          idx_v.at[pl.ds(1, K - 1)], sir)
    c0.wait()

    def islice(kk, ci):
      return idx_v.at[kk, pl.ds(ci * C, C)]

    def plain(ci):
      pltpu.async_copy(tab_hbm.at[0].at[islice(0, ci)], acc[ci % NB], sg[ci])

    def fire_out(ci):
      pltpu.async_copy(acc[ci % NB],
                       out_hbm.at[pl.ds(wid * PPW + ci * C, C)], so[ci])

    def drain_out(ci):
      pltpu.make_async_copy(acc[ci % NB], out_hbm.at[pl.ds(0, C)],
                            so[ci]).wait()

    for ci in range(PF):
      plain(ci)
    cr.wait()
    prev_adds = None
    drained = 0
    for ci in range(NCH):
      pltpu.make_async_copy(
          tab_hbm.at[0].at[islice(0, ci)], acc[ci % NB], sg[ci]).wait()
      adds = [
          pltpu.async_copy(tab_hbm.at[kk].at[islice(kk, ci)], acc[ci % NB],
                           sa[ci], add=True)
          for kk in range(1, K)
      ]
      if prev_adds is not None:
        for cp in prev_adds:
          cp.wait()
        fire_out(ci - 1)
      prev_adds = adds
      if ci + PF < NCH:
        if ci >= 2:
          drain_out(ci - 2)
          drained = ci - 1
        plain(ci + PF)
    for cp in prev_adds:
      cp.wait()
    fire_out(NCH - 1)
    for ci in range(drained, NCH):
      drain_out(ci)

  return k


_sc_kernel = _make_kernel()


def kernel(codes, tables):
  out = _sc_kernel(codes.astype(jnp.int32), tables)
  return out.reshape(B, T, D)
